# Initial kernel scaffold; baseline (speedup 1.0000x reference)
import functools

import jax
import jax.numpy as jnp
from jax import lax
from jax.experimental import pallas as pl
from jax.experimental.pallas import tpu as pltpu
from jax.experimental.pallas import tpu_sc as plsc

D = 128
K = 128
NC = 2
NS = 16
NW = NC * NS
SELF_LOOP_TOKEN = 4


def _sc_mesh():
    return plsc.VectorSubcoreMesh(
        core_axis_name="c", subcore_axis_name="s", num_cores=NC, num_subcores=NS
    )


def _make_scatter_kernel(npad, trows):
    rpt = npad // NS

    @functools.partial(
        pl.kernel,
        out_type=jax.ShapeDtypeStruct((NC, npad, D), jnp.float32),
        mesh=_sc_mesh(),
        scratch_types=dict(
            agg_sh=pltpu.VMEM_SHARED((npad, D), jnp.float32),
            sidx=pltpu.VMEM((K,), jnp.int32),
            didx=pltpu.VMEM((K,), jnp.int32),
            rows=pltpu.VMEM((K, D), jnp.float32),
            sem=pltpu.SemaphoreType.DMA,
        ),
    )
    def scatter(h_hbm, srcR, dstR, zrows, out, agg_sh, sidx, didx, rows, sem):
        c = lax.axis_index("c")
        s = lax.axis_index("s")
        wid = c * NS + s
        pltpu.sync_copy(zrows, agg_sh.at[pl.ds(s * rpt, rpt)])
        plsc.subcore_barrier()

        def body(b, carry):
            r = wid * trows + b
            pltpu.sync_copy(srcR.at[r], sidx)
            pltpu.sync_copy(dstR.at[r], didx)
            pltpu.async_copy(h_hbm.at[sidx], rows, sem).wait()
            pltpu.sync_copy(rows, agg_sh.at[didx], add=True)
            return carry

        lax.fori_loop(0, trows, body, 0)
        plsc.subcore_barrier()
        pltpu.sync_copy(agg_sh.at[pl.ds(s * rpt, rpt)], out.at[c].at[pl.ds(s * rpt, rpt)])

    return scatter


def _make_counts_kernel(npad, trows):
    rpt = npad // NS

    @functools.partial(
        pl.kernel,
        out_type=jax.ShapeDtypeStruct((NC, npad, 16), jnp.float32),
        mesh=_sc_mesh(),
        scratch_types=dict(
            cnt_sh=pltpu.VMEM_SHARED((npad, 16), jnp.float32),
            didx=pltpu.VMEM((K,), jnp.int32),
            ea0b=pltpu.VMEM((K,), jnp.int32),
            ea1b=pltpu.VMEM((K,), jnp.int32),
            oh=pltpu.VMEM((K, 16), jnp.float32),
        ),
    )
    def counts(dstR, ea0R, ea1R, z16, out, cnt_sh, didx, ea0b, ea1b, oh):
        c = lax.axis_index("c")
        s = lax.axis_index("s")
        wid = c * NS + s
        pltpu.sync_copy(z16.at[pl.ds(0, rpt)], cnt_sh.at[pl.ds(s * rpt, rpt)])
        pltpu.sync_copy(z16.at[pl.ds(0, K)], oh)
        plsc.subcore_barrier()

        ones = jnp.full((16,), 1.0, jnp.float32)
        zeros = jnp.zeros((16,), jnp.float32)

        def body(b, carry):
            r = wid * trows + b
            pltpu.sync_copy(dstR.at[r], didx)
            pltpu.sync_copy(ea0R.at[r], ea0b)
            pltpu.sync_copy(ea1R.at[r], ea1b)
            a0s = []
            a1s = []
            for g in range(K // 16):
                eids = lax.iota(jnp.int32, 16) + (g * 16)
                a0 = ea0b[pl.ds(g * 16, 16)]
                a1 = ea1b[pl.ds(g * 16, 16)] + 8
                plsc.store_scatter(oh, [eids, a0], ones)
                plsc.store_scatter(oh, [eids, a1], ones)
                a0s.append(a0)
                a1s.append(a1)
            pltpu.sync_copy(oh, cnt_sh.at[didx], add=True)
            for g in range(K // 16):
                eids = lax.iota(jnp.int32, 16) + (g * 16)
                plsc.store_scatter(oh, [eids, a0s[g]], zeros)
                plsc.store_scatter(oh, [eids, a1s[g]], zeros)
            return carry

        lax.fori_loop(0, trows, body, 0)
        plsc.subcore_barrier()
        pltpu.sync_copy(cnt_sh.at[pl.ds(s * rpt, rpt)], out.at[c].at[pl.ds(s * rpt, rpt)])

    return counts


def _tc_embed(npad, n):
    def body(x0_ref, x1_ref, e1_ref, e2_ref, o_ref):
        iot = lax.broadcasted_iota(jnp.int32, (1, 128), 1)
        oh0 = (x0_ref[...] == iot).astype(jnp.float32)
        oh1 = (x1_ref[...] == iot).astype(jnp.float32)
        h0 = jnp.dot(oh0, e1_ref[...], preferred_element_type=jnp.float32)
        h0 = h0 + jnp.dot(oh1, e2_ref[...], preferred_element_type=jnp.float32)
        rid = lax.broadcasted_iota(jnp.int32, (npad, 1), 0)
        o_ref[...] = jnp.where(rid < n, h0, 0.0)

    return pl.pallas_call(
        body, out_shape=jax.ShapeDtypeStruct((npad, D), jnp.float32)
    )


def _tc_layer(npad, n, last):
    def body(aggP_ref, h_ref, cnt_ref, ee_ref, sr_ref, w1_ref, b1_ref,
             w2_ref, b2_ref, gm_ref, bt_ref, o_ref):
        cnt = cnt_ref[0] + cnt_ref[1]
        edge_part = jnp.dot(cnt, ee_ref[...], preferred_element_type=jnp.float32)
        agg = aggP_ref[0] + aggP_ref[1] + h_ref[...] + edge_part + sr_ref[...]
        hid = jnp.dot(agg, w1_ref[...], preferred_element_type=jnp.float32)
        hid = jnp.maximum(hid + b1_ref[...], 0.0)
        out = jnp.dot(hid, w2_ref[...], preferred_element_type=jnp.float32)
        out = out + b2_ref[...]
        rid = lax.broadcasted_iota(jnp.int32, (npad, 1), 0)
        rmask = rid < n
        outm = jnp.where(rmask, out, 0.0)
        mean = jnp.sum(outm, axis=0, keepdims=True) / n
        dlt = jnp.where(rmask, out - mean, 0.0)
        var = jnp.sum(dlt * dlt, axis=0, keepdims=True) / n
        hn = (out - mean) * lax.rsqrt(var + 1e-5) * gm_ref[...] + bt_ref[...]
        if not last:
            hn = jnp.maximum(hn, 0.0)
        o_ref[...] = jnp.where(rmask, hn, 0.0)

    return pl.pallas_call(
        body, out_shape=jax.ShapeDtypeStruct((npad, D), jnp.float32)
    )


def kernel(x, edge_index, edge_attr, x_emb1, x_emb2, ee1, ee2, W1, b1, W2, b2,
           gamma, beta):
    n = x.shape[0]
    e = edge_index.shape[1]
    L = W1.shape[0]
    npad = ((n + NW * 8 - 1) // (NW * 8)) * (NW * 8) + NW * 8
    erows = -(-e // K)
    erows_pad = ((erows + NW - 1) // NW) * NW
    trows = erows_pad // NW
    epad = erows_pad * K
    rpt = npad // NS

    src = edge_index[0].astype(jnp.int32)
    dst = edge_index[1].astype(jnp.int32)
    ea0 = edge_attr[:, 0].astype(jnp.int32)
    ea1 = edge_attr[:, 1].astype(jnp.int32)
    padi = jnp.full((epad - e,), n, jnp.int32)
    pad0 = jnp.zeros((epad - e,), jnp.int32)
    srcR = jnp.concatenate([src, padi]).reshape(erows_pad, K)
    dstR = jnp.concatenate([dst, padi]).reshape(erows_pad, K)
    ea0R = jnp.concatenate([ea0, pad0]).reshape(erows_pad, K)
    ea1R = jnp.concatenate([ea1, pad0]).reshape(erows_pad, K)

    x0p = jnp.zeros((npad, 1), jnp.int32).at[:n, 0].set(x[:, 0].astype(jnp.int32))
    x1p = jnp.zeros((npad, 1), jnp.int32).at[:n, 0].set(x[:, 1].astype(jnp.int32))
    e1p = jnp.zeros((128, D), jnp.float32).at[: x_emb1.shape[0]].set(x_emb1)
    e2p = jnp.zeros((128, D), jnp.float32).at[: x_emb2.shape[0]].set(x_emb2)

    nb1 = ee1.shape[1]
    nb2 = ee2.shape[1]
    EE = jnp.zeros((L, 16, D), jnp.float32)
    EE = EE.at[:, :nb1].set(ee1).at[:, 8 : 8 + nb2].set(ee2)
    selfrow = (ee1[:, SELF_LOOP_TOKEN, :] + ee2[:, 0, :]).reshape(L, 1, D)

    zrows = jnp.zeros((rpt, D), jnp.float32)
    z16 = jnp.zeros((max(rpt, K), 16), jnp.float32)

    scatter = _make_scatter_kernel(npad, trows)
    counts_k = _make_counts_kernel(npad, trows)
    embed = _tc_embed(npad, n)

    h = embed(x0p, x1p, e1p, e2p)
    cnt2 = counts_k(dstR, ea0R, ea1R, z16)
    for l in range(L):
        aggP = scatter(h, srcR, dstR, zrows)
        layer = _tc_layer(npad, n, last=(l == L - 1))
        h = layer(aggP, h, cnt2, EE[l], selfrow[l],
                  W1[l], b1[l].reshape(1, -1), W2[l], b2[l].reshape(1, -1),
                  gamma[l].reshape(1, -1), beta[l].reshape(1, -1))
    return h[:n]


# speedup vs baseline: 3.8556x; 3.8556x over previous
import functools

import jax
import jax.numpy as jnp
import numpy as np
from jax import lax
from jax.experimental import pallas as pl
from jax.experimental.pallas import tpu as pltpu
from jax.experimental.pallas import tpu_sc as plsc

D = 128
K = 128
NC = 2
NS = 16
NW = NC * NS
SELF_LOOP_TOKEN = 4


def _sc_mesh():
    return plsc.VectorSubcoreMesh(
        core_axis_name="c", subcore_axis_name="s", num_cores=NC, num_subcores=NS
    )


def _make_scatter_kernel(npad, trows, width):
    rpt = npad // NS

    @functools.partial(
        pl.kernel,
        out_type=jax.ShapeDtypeStruct((NC, npad, width), jnp.float32),
        mesh=_sc_mesh(),
        scratch_types=dict(
            agg_sh=pltpu.VMEM_SHARED((npad, width), jnp.float32),
            sidx=pltpu.VMEM((K,), jnp.int32),
            didx=pltpu.VMEM((K,), jnp.int32),
            rows=pltpu.VMEM((K, width), jnp.float32),
            sem=pltpu.SemaphoreType.DMA,
        ),
    )
    def scatter(tbl_hbm, srcR, dstR, zrows, out, agg_sh, sidx, didx, rows, sem):
        c = lax.axis_index("c")
        s = lax.axis_index("s")
        wid = c * NS + s
        pltpu.sync_copy(zrows, agg_sh.at[pl.ds(s * rpt, rpt)])
        plsc.subcore_barrier()

        def body(b, carry):
            r = wid * trows + b
            pltpu.sync_copy(srcR.at[r], sidx)
            pltpu.sync_copy(dstR.at[r], didx)
            pltpu.async_copy(tbl_hbm.at[sidx], rows, sem).wait()
            pltpu.sync_copy(rows, agg_sh.at[didx], add=True)
            return carry

        lax.fori_loop(0, trows, body, 0)
        plsc.subcore_barrier()
        pltpu.sync_copy(agg_sh.at[pl.ds(s * rpt, rpt)],
                        out.at[c].at[pl.ds(s * rpt, rpt)])

    return scatter


def _tc_embed(npad, n):
    def body(x0_ref, x1_ref, e1_ref, e2_ref, o_ref):
        iot = lax.broadcasted_iota(jnp.int32, (1, 128), 1)
        oh0 = (x0_ref[...] == iot).astype(jnp.float32)
        oh1 = (x1_ref[...] == iot).astype(jnp.float32)
        h0 = jnp.dot(oh0, e1_ref[...], preferred_element_type=jnp.float32, precision=lax.Precision.HIGHEST)
        h0 = h0 + jnp.dot(oh1, e2_ref[...], preferred_element_type=jnp.float32, precision=lax.Precision.HIGHEST)
        rid = lax.broadcasted_iota(jnp.int32, (npad, 1), 0)
        o_ref[...] = jnp.where(rid < n, h0, 0.0)

    return pl.pallas_call(
        body, out_shape=jax.ShapeDtypeStruct((npad, D), jnp.float32)
    )


def _tc_layer(npad, n, last):
    def body(aggP_ref, h_ref, cnt_ref, ee_ref, sr_ref, w1_ref, b1_ref,
             w2_ref, b2_ref, gm_ref, bt_ref, o_ref):
        cnt = cnt_ref[0] + cnt_ref[1]
        edge_part = jnp.dot(cnt, ee_ref[...], preferred_element_type=jnp.float32, precision=lax.Precision.HIGHEST)
        agg = aggP_ref[0] + aggP_ref[1] + h_ref[...] + edge_part + sr_ref[...]
        hid = jnp.dot(agg.astype(jnp.bfloat16), w1_ref[...].astype(jnp.bfloat16),
                      preferred_element_type=jnp.float32)
        hid = jnp.maximum(hid + b1_ref[...], 0.0)
        out = jnp.dot(hid.astype(jnp.bfloat16), w2_ref[...].astype(jnp.bfloat16),
                      preferred_element_type=jnp.float32)
        out = out + b2_ref[...]
        rid = lax.broadcasted_iota(jnp.int32, (npad, 1), 0)
        rmask = rid < n
        outm = jnp.where(rmask, out, 0.0)
        mean = jnp.sum(outm, axis=0, keepdims=True) / n
        dlt = jnp.where(rmask, out - mean, 0.0)
        var = jnp.sum(dlt * dlt, axis=0, keepdims=True) / n
        hn = (out - mean) / jnp.sqrt(var + 1e-5) * gm_ref[...] + bt_ref[...]
        if not last:
            hn = jnp.maximum(hn, 0.0)
        o_ref[...] = jnp.where(rmask, hn, 0.0)

    return pl.pallas_call(
        body, out_shape=jax.ShapeDtypeStruct((npad, D), jnp.float32)
    )


def kernel(x, edge_index, edge_attr, x_emb1, x_emb2, ee1, ee2, W1, b1, W2, b2,
           gamma, beta):
    n = x.shape[0]
    e = edge_index.shape[1]
    L = W1.shape[0]
    npad = ((n + NW * 8 - 1) // (NW * 8)) * (NW * 8) + NW * 8
    erows = -(-e // K)
    erows_pad = ((erows + NW - 1) // NW) * NW
    trows = erows_pad // NW
    epad = erows_pad * K
    rpt = npad // NS

    src = edge_index[0].astype(jnp.int32)
    dst = edge_index[1].astype(jnp.int32)
    ea0 = edge_attr[:, 0].astype(jnp.int32)
    ea1 = edge_attr[:, 1].astype(jnp.int32)
    nb2 = ee2.shape[1]
    ecode = ea0 * nb2 + ea1
    padi = jnp.full((epad - e,), n, jnp.int32)
    padc = jnp.full((epad - e,), 31, jnp.int32)
    srcR = jnp.concatenate([src, padi]).reshape(erows_pad, K)
    dstR = jnp.concatenate([dst, padi]).reshape(erows_pad, K)
    ecodeR = jnp.concatenate([ecode, padc]).reshape(erows_pad, K)

    oh_np = np.zeros((32, D), np.float32)
    for a0 in range(ee1.shape[1]):
        for a1 in range(nb2):
            oh_np[a0 * nb2 + a1, a0] += 1.0
            oh_np[a0 * nb2 + a1, 8 + a1] += 1.0
    ohlut = jnp.asarray(oh_np)

    x0p = jnp.zeros((npad, 1), jnp.int32).at[:n, 0].set(x[:, 0].astype(jnp.int32))
    x1p = jnp.zeros((npad, 1), jnp.int32).at[:n, 0].set(x[:, 1].astype(jnp.int32))
    e1p = jnp.zeros((128, D), jnp.float32).at[: x_emb1.shape[0]].set(x_emb1)
    e2p = jnp.zeros((128, D), jnp.float32).at[: x_emb2.shape[0]].set(x_emb2)

    nb1 = ee1.shape[1]
    EE = jnp.zeros((L, 16, D), jnp.float32)
    EE = EE.at[:, :nb1].set(ee1).at[:, 8 : 8 + nb2].set(ee2)
    selfrow = (ee1[:, SELF_LOOP_TOKEN, :] + ee2[:, 0, :]).reshape(L, 1, D)

    zrows = jnp.zeros((rpt, D), jnp.float32)

    scatter = _make_scatter_kernel(npad, trows, D)
    embed = _tc_embed(npad, n)

    h = embed(x0p, x1p, e1p, e2p)
    cnt2 = scatter(ohlut, ecodeR, dstR, zrows)[:, :, :16]
    for l in range(L):
        aggP = scatter(h, srcR, dstR, zrows)
        layer = _tc_layer(npad, n, last=(l == L - 1))
        h = layer(aggP, h, cnt2, EE[l], selfrow[l],
                  W1[l], b1[l].reshape(1, -1), W2[l], b2[l].reshape(1, -1),
                  gamma[l].reshape(1, -1), beta[l].reshape(1, -1))
    return h[:n]
